# baseline (speedup 1.0000x reference)
import functools

import jax
import jax.numpy as jnp
from jax import lax
from jax.experimental import pallas as pl
from jax.experimental.pallas import tpu as pltpu
from jax.experimental.pallas import tpu_sc as plsc

_F = 26
_D = 16
_B = 4096
_V = 26000
_PAIRS = [(f, g) for f in range(_F - 1) for g in range(f + 1, _F)]
_NP = len(_PAIRS)
_INTER = _NP * _D
_HPAD = 5248
_ROW = 512

_NW = 32
_BPW = _B // _NW
_NB = 2
_NBW = 8
_NCHUNK = _BPW // _NB
_IDXP = 64


def _sc_interactions(embT, idx):
    mesh = plsc.VectorSubcoreMesh(core_axis_name="c", subcore_axis_name="s")

    @functools.partial(
        pl.kernel,
        mesh=mesh,
        out_type=[
            jax.ShapeDtypeStruct((_B, _HPAD), jnp.float32),
            jax.ShapeDtypeStruct((_B, 16), jnp.float32),
        ],
        scratch_types=[
            pltpu.VMEM((_NCHUNK * _IDXP,), jnp.int32),
            pltpu.VMEM((2 * _IDXP, _ROW), jnp.float32),
            pltpu.VMEM((_NBW, _HPAD), jnp.float32),
            pltpu.VMEM((_NBW, 16), jnp.float32),
            pltpu.SemaphoreType.DMA,
        ],
    )
    def k(embT_hbm, idx_hbm, h_hbm, fo_hbm, idx_v, rows_v, h_v, fo_v, sem):
        wid = lax.axis_index("s") * 2 + lax.axis_index("c")
        ibase = wid * _NCHUNK * _IDXP

        zeros16 = jnp.zeros((16,), jnp.float32)
        for bl in range(_NBW):
            for c in range(_INTER, _HPAD, 16):
                h_v[bl, pl.ds(c, 16)] = zeros16

        pltpu.sync_copy(idx_hbm.at[pl.ds(ibase, _NCHUNK * _IDXP)], idx_v)

        def _gather(chunk, slot):
            src = embT_hbm.at[idx_v.at[pl.ds(chunk * _IDXP, _IDXP)]]
            dst = rows_v.at[pl.ds(pl.multiple_of(slot * _IDXP, 8), _IDXP)]
            return src, dst

        s0, d0 = _gather(0, 0)
        pltpu.async_copy(s0, d0, sem)

        def step(chunk, carry):
            slot = lax.rem(chunk, 2)
            nslot = 1 - slot

            @pl.when(chunk < _NCHUNK - 1)
            def _prefetch():
                src, dst = _gather(chunk + 1, nslot)
                pltpu.async_copy(src, dst, sem)

            src, dst = _gather(chunk, slot)
            pltpu.make_async_copy(src, dst, sem).wait()

            rbase = slot * _IDXP
            hbase = lax.rem(chunk, 4) * _NB

            def elem_body(bl, c2):
                r0 = rbase + bl * _F
                hrow = hbase + bl
                for q in range(0, _NP, 5):
                    grp = _PAIRS[q:q + 5]
                    avs = [rows_v[r0 + g, pl.ds(f * _D, 16)] for f, g in grp]
                    bvs = [rows_v[r0 + f, pl.ds(g * _D, 16)] for f, g in grp]
                    pvs = [a * b for a, b in zip(avs, bvs)]
                    for i, pv in enumerate(pvs):
                        h_v[hrow, pl.ds((q + i) * _D, 16)] = pv
                accs = [rows_v[r0 + g, pl.ds(_F * _D, 16)]
                        for g in range(_F)]
                while len(accs) > 1:
                    accs = [accs[i] + accs[i + 1] if i + 1 < len(accs)
                            else accs[i] for i in range(0, len(accs), 2)]
                fo_v[hrow, :] = accs[0]
                return c2

            lax.fori_loop(0, _NB, elem_body, 0)

            @pl.when(lax.rem(chunk, 4) == 3)
            def _write():
                b0 = pl.multiple_of(wid * _BPW + (chunk - 3) * _NB, _NBW)
                pltpu.sync_copy(h_v, h_hbm.at[pl.ds(b0, _NBW)])
                pltpu.sync_copy(fo_v, fo_hbm.at[pl.ds(b0, _NBW)])

            return carry

        lax.fori_loop(0, _NCHUNK, step, 0)

    return k(embT, idx)


def _tc_mlp(h, fo, W1p, b1, W2, b2, W3, b3):
    bt = 256

    def body(h_ref, fo_ref, w1_ref, b1_ref, w2_ref, b2_ref, w3_ref, b3_ref, out_ref):
        y = jnp.dot(h_ref[...], w1_ref[...], preferred_element_type=jnp.float32)
        y = jnp.maximum(y + b1_ref[...], 0.0)
        y = jnp.dot(y, w2_ref[...], preferred_element_type=jnp.float32)
        y = jnp.maximum(y + b2_ref[...], 0.0)
        z = jnp.dot(y, w3_ref[...], preferred_element_type=jnp.float32)
        out_ref[...] = z + b3_ref[...] + fo_ref[:, :1]

    return pl.pallas_call(
        body,
        grid=(_B // bt,),
        in_specs=[
            pl.BlockSpec((bt, _HPAD), lambda i: (i, 0)),
            pl.BlockSpec((bt, 16), lambda i: (i, 0)),
            pl.BlockSpec((_HPAD, 64), lambda i: (0, 0)),
            pl.BlockSpec((1, 64), lambda i: (0, 0)),
            pl.BlockSpec((64, 32), lambda i: (0, 0)),
            pl.BlockSpec((1, 32), lambda i: (0, 0)),
            pl.BlockSpec((32, 1), lambda i: (0, 0)),
            pl.BlockSpec((1, 1), lambda i: (0, 0)),
        ],
        out_specs=pl.BlockSpec((bt, 1), lambda i: (i, 0)),
        out_shape=jax.ShapeDtypeStruct((_B, 1), jnp.float32),
    )(h, fo, W1p, b1, W2, b2, W3, b3)


def kernel(x, emb, w_lin, b_lin, W1, b1, W2, b2, W3, b3):
    offs = (jnp.arange(_F, dtype=x.dtype) * 1000)[None, :]
    idx = jnp.pad((x + offs).reshape(_B // _NB, _NB * _F),
                  ((0, 0), (0, _IDXP - _NB * _F))).reshape(-1)
    embT = jnp.concatenate(
        [
            emb.transpose(1, 0, 2).reshape(_V, _F * _D),
            w_lin,
            jnp.zeros((_V, _ROW - _F * _D - 1), jnp.float32),
        ],
        axis=1,
    )
    W1p = jnp.concatenate([W1, jnp.zeros((_HPAD - _INTER, 64), jnp.float32)], axis=0)
    h, fo = _sc_interactions(embT, idx)
    out = _tc_mlp(h, fo, W1p, b1.reshape(1, 64), W2, b2.reshape(1, 32),
                  W3, b3.reshape(1, 1))
    return out[:, 0] + b_lin[0]


# speedup vs baseline: 3.2364x; 3.2364x over previous
import functools

import jax
import jax.numpy as jnp
from jax import lax
from jax.experimental import pallas as pl
from jax.experimental.pallas import tpu as pltpu
from jax.experimental.pallas import tpu_sc as plsc

_F = 26
_D = 16
_B = 4096
_V = 26000
_PAIRS = [(f, g) for f in range(_F - 1) for g in range(f + 1, _F)]
_NP = len(_PAIRS)
_INTER = _NP * _D
_HPAD = 5248
_ROW = 512

_NW = 32
_BPW = _B // _NW
_NB = 2
_NBW = 8
_NCHUNK = _BPW // _NB
_IDXP = 64


def _sc_interactions(embT, idx):
    mesh = plsc.VectorSubcoreMesh(core_axis_name="c", subcore_axis_name="s")

    @functools.partial(
        pl.kernel,
        mesh=mesh,
        out_type=[
            jax.ShapeDtypeStruct((_B, _HPAD), jnp.float32),
            jax.ShapeDtypeStruct((_B, 16), jnp.float32),
        ],
        scratch_types=[
            pltpu.VMEM((_NCHUNK * _IDXP,), jnp.int32),
            pltpu.VMEM((2 * _IDXP, _ROW), jnp.float32),
            pltpu.VMEM((_NBW, _HPAD), jnp.float32),
            pltpu.VMEM((_NBW, 16), jnp.float32),
            pltpu.SemaphoreType.DMA,
        ],
    )
    def k(embT_hbm, idx_hbm, h_hbm, fo_hbm, idx_v, rows_v, h_v, fo_v, sem):
        wid = lax.axis_index("s") * 2 + lax.axis_index("c")
        ibase = wid * _NCHUNK * _IDXP

        zeros16 = jnp.zeros((16,), jnp.float32)
        for bl in range(_NBW):
            for c in range(_INTER, _HPAD, 16):
                h_v[bl, pl.ds(c, 16)] = zeros16

        pltpu.sync_copy(idx_hbm.at[pl.ds(ibase, _NCHUNK * _IDXP)], idx_v)

        def _gather(chunk, slot, lo, n):
            src = embT_hbm.at[idx_v.at[pl.ds(chunk * _IDXP + lo, n)]]
            dst = rows_v.at[
                pl.ds(pl.multiple_of(slot * _IDXP, 8) + lo, n)]
            return src, dst

        s0, d0 = _gather(0, 0, 0, _IDXP)
        pltpu.async_copy(s0, d0, sem)

        def step(chunk, carry):
            slot = lax.rem(chunk, 2)
            nslot = 1 - slot

            @pl.when(chunk < _NCHUNK - 1)
            def _prefetch():
                src, dst = _gather(chunk + 1, nslot, 0, _IDXP)
                pltpu.async_copy(src, dst, sem)

            src, dst = _gather(chunk, slot, 0, _IDXP)
            pltpu.make_async_copy(src, dst, sem).wait()

            rbase = slot * _IDXP
            hbase = lax.rem(chunk, 4) * _NB

            def elem_body(bl, c2):
                r0 = rbase + bl * _F
                hrow = hbase + bl
                for q in range(0, _NP, 5):
                    grp = _PAIRS[q:q + 5]
                    avs = [rows_v[r0 + g, pl.ds(f * _D, 16)] for f, g in grp]
                    bvs = [rows_v[r0 + f, pl.ds(g * _D, 16)] for f, g in grp]
                    pvs = [a * b for a, b in zip(avs, bvs)]
                    for i, pv in enumerate(pvs):
                        h_v[hrow, pl.ds((q + i) * _D, 16)] = pv
                accs = [rows_v[r0 + g, pl.ds(_F * _D, 16)]
                        for g in range(_F)]
                while len(accs) > 1:
                    accs = [accs[i] + accs[i + 1] if i + 1 < len(accs)
                            else accs[i] for i in range(0, len(accs), 2)]
                fo_v[hrow, :] = accs[0]
                return c2

            lax.fori_loop(0, _NB, elem_body, 0)

            @pl.when(lax.rem(chunk, 4) == 3)
            def _write():
                b0 = pl.multiple_of(wid * _BPW + (chunk - 3) * _NB, _NBW)
                pltpu.sync_copy(h_v, h_hbm.at[pl.ds(b0, _NBW)])
                pltpu.sync_copy(fo_v, fo_hbm.at[pl.ds(b0, _NBW)])

            return carry

        lax.fori_loop(0, _NCHUNK, step, 0)

    return k(embT, idx)


def _tc_mlp(h, fo, W1p, b1, W2, b2, W3, b3):
    bt = 256

    def body(h_ref, fo_ref, w1_ref, b1_ref, w2_ref, b2_ref, w3_ref, b3_ref, out_ref):
        y = jnp.dot(h_ref[...], w1_ref[...], preferred_element_type=jnp.float32)
        y = jnp.maximum(y + b1_ref[...], 0.0)
        y = jnp.dot(y, w2_ref[...], preferred_element_type=jnp.float32)
        y = jnp.maximum(y + b2_ref[...], 0.0)
        z = jnp.dot(y, w3_ref[...], preferred_element_type=jnp.float32)
        out_ref[...] = z + b3_ref[...] + fo_ref[:, :1]

    return pl.pallas_call(
        body,
        grid=(_B // bt,),
        in_specs=[
            pl.BlockSpec((bt, _HPAD), lambda i: (i, 0)),
            pl.BlockSpec((bt, 16), lambda i: (i, 0)),
            pl.BlockSpec((_HPAD, 64), lambda i: (0, 0)),
            pl.BlockSpec((1, 64), lambda i: (0, 0)),
            pl.BlockSpec((64, 32), lambda i: (0, 0)),
            pl.BlockSpec((1, 32), lambda i: (0, 0)),
            pl.BlockSpec((32, 1), lambda i: (0, 0)),
            pl.BlockSpec((1, 1), lambda i: (0, 0)),
        ],
        out_specs=pl.BlockSpec((bt, 1), lambda i: (i, 0)),
        out_shape=jax.ShapeDtypeStruct((_B, 1), jnp.float32),
    )(h, fo, W1p, b1, W2, b2, W3, b3)


def kernel(x, emb, w_lin, b_lin, W1, b1, W2, b2, W3, b3):
    offs = (jnp.arange(_F, dtype=x.dtype) * 1000)[None, :]
    nchunks = _B // _NB
    npad = _IDXP - _NB * _F
    filler = (jnp.arange(nchunks * npad, dtype=jnp.int32)
              .reshape(nchunks, npad) * 997) % _V
    idx = jnp.concatenate(
        [(x + offs).reshape(nchunks, _NB * _F), filler], axis=1).reshape(-1)
    embT = jnp.concatenate(
        [
            emb.transpose(1, 0, 2).reshape(_V, _F * _D),
            w_lin,
            jnp.zeros((_V, _ROW - _F * _D - 1), jnp.float32),
        ],
        axis=1,
    )
    W1p = jnp.concatenate([W1, jnp.zeros((_HPAD - _INTER, 64), jnp.float32)], axis=0)
    h, fo = _sc_interactions(embT, idx)
    out = _tc_mlp(h, fo, W1p, b1.reshape(1, 64), W2, b2.reshape(1, 32),
                  W3, b3.reshape(1, 1))
    return out[:, 0] + b_lin[0]
